# baseline (speedup 1.0000x reference)
import functools

import jax
import jax.numpy as jnp
from jax import lax
from jax.experimental import pallas as pl
from jax.experimental.pallas import tpu as pltpu
from jax.experimental.pallas import tpu_sc as plsc

BATCH = 16384
EMBED = 32
NC = 2
NS = 16
NW = NC * NS
B_PER_W = BATCH // NW
NBUF = 16
ROUNDS = B_PER_W // 16
CHUNK = 128
NCHUNK = B_PER_W // CHUNK


def _ring_control(sorted_idx):
    iota = jnp.arange(BATCH, dtype=jnp.int32)
    pan = sorted_idx >> 7
    kmod = iota & (B_PER_W - 1)
    w = iota >> 9
    prev = jnp.concatenate([pan[:1] - 1, pan[:-1]])
    fetch = (pan != prev) | (kmod == 0)
    m = (jnp.cumsum(fetch.reshape(NW, B_PER_W), axis=1)
         .reshape(-1).astype(jnp.int32) - 1)
    npan = m.reshape(NW, B_PER_W)[:, -1] + 1
    slot = m & (NBUF - 1)
    ps = jnp.full((BATCH,), -1, jnp.int32).at[w * B_PER_W + m].set(pan << 7)
    nxt_fetch = jnp.concatenate([fetch[1:], jnp.ones((1,), fetch.dtype)])
    q2 = m + NBUF
    issue = nxt_fetch & (q2 < npan[w])
    parr = jnp.take(ps, jnp.clip(w * B_PER_W + q2, 0, BATCH - 1))
    return (sorted_idx, fetch.astype(jnp.int32), slot,
            issue.astype(jnp.int32), parr, ps)


def _gather_sorted(tbl, ctl, out_ref, base, panels, stg, sems):
    sv, fv, slv, isv, pav, psv = ctl
    rows0 = lax.iota(jnp.int32, 16)
    rows1 = rows0 + 16

    pstart = psv[pl.ds(0, 16)]
    for k in range(NBUF):
        @pl.when(pstart[k] >= 0)
        def _():
            st = pl.multiple_of(pstart[k], 128)
            pltpu.async_copy(tbl.at[:, pl.ds(st, 128)], panels.at[k],
                             sems.at[k])

    def round_body(g, carry):
        off = pl.multiple_of(g * 16, 8)
        v_s = sv[pl.ds(off, 16)]
        v_f = fv[pl.ds(off, 16)]
        v_sl = slv[pl.ds(off, 16)]
        v_is = isv[pl.ds(off, 16)]
        v_pa = pav[pl.ds(off, 16)]
        for k in range(16):
            sl = v_sl[k]

            @pl.when(v_f[k] != 0)
            def _():
                pltpu.make_async_copy(tbl.at[:, pl.ds(0, 128)],
                                      panels.at[sl], sems.at[sl]).wait()

            r = v_s[k]
            col = jnp.zeros((16,), jnp.int32) + (r & 127)
            v0 = plsc.load_gather(panels.at[sl], [rows0, col])
            v1 = plsc.load_gather(panels.at[sl], [rows1, col])
            row = (g * 16 + k) & 127
            stg[row, pl.ds(0, 16)] = v0
            stg[row, pl.ds(16, 16)] = v1

            @pl.when(v_is[k] != 0)
            def _():
                st = pl.multiple_of(v_pa[k], 128)
                pltpu.async_copy(tbl.at[:, pl.ds(st, 128)], panels.at[sl],
                                 sems.at[sl])

        @pl.when(lax.rem(g, 8) == 7)
        def _():
            off2 = pl.multiple_of(base + ((g // 8) << 7), 128)
            pltpu.sync_copy(stg, out_ref.at[pl.ds(off2, 128)])

        return carry

    lax.fori_loop(0, ROUNDS, round_body, 0)


def _sc_gather_half_body(*args):
    ctl_hbm = args[0:6]
    tblA, tblB = args[6:8]
    outA, outB = args[8:10]
    ctl = list(args[10:16])
    (panels, stg, sems) = args[16:19]
    c = lax.axis_index("c")
    s = lax.axis_index("s")
    wid = s * NC + c
    base = pl.multiple_of(wid * B_PER_W, 128)
    for a in range(6):
        pltpu.sync_copy(ctl_hbm[a].at[pl.ds(base, B_PER_W)], ctl[a])
    _gather_sorted(tblA, ctl, outA, base, panels, stg, sems)
    _gather_sorted(tblB, ctl, outB, base, panels, stg, sems)


@functools.cache
def _sc_gather_half():
  return pl.kernel(
    _sc_gather_half_body,
    out_type=[jax.ShapeDtypeStruct((BATCH, EMBED), jnp.float32)] * 2,
    mesh=plsc.VectorSubcoreMesh(core_axis_name="c", subcore_axis_name="s",
                                num_cores=NC, num_subcores=NS),
    scratch_types=[pltpu.VMEM((B_PER_W,), jnp.int32)] * 6 + [
        pltpu.VMEM((NBUF, EMBED, 128), jnp.float32),
        pltpu.VMEM((128, EMBED), jnp.float32),
        pltpu.SemaphoreType.DMA((NBUF,)),
    ],
    compiler_params=pltpu.CompilerParams(use_tc_tiling_on_sc=True,
                                         disable_bounds_checks=True,
                                         needs_layout_passes=False),
  )


def _sc_unperm_body(pu_hbm, pi_hbm, sug, sig, sum_, sim,
                    out_ug, out_ig, out_um, out_im,
                    uidx, iidx, bug, big, bum, bim,
                    sem0, sem1, sem2, sem3):
    c = lax.axis_index("c")
    s = lax.axis_index("s")
    wid = s * NC + c
    base = pl.multiple_of(wid * B_PER_W, 128)
    pltpu.sync_copy(pu_hbm.at[wid], uidx)
    pltpu.sync_copy(pi_hbm.at[wid], iidx)
    span = pl.ds(base, B_PER_W)
    pltpu.sync_copy(sug.at[span], bug)
    pltpu.sync_copy(sig.at[span], big)
    pltpu.sync_copy(sum_.at[span], bum)
    pltpu.sync_copy(sim.at[span], bim)
    cps = []
    for j in range(NCHUNK):
        src = pl.ds(j * CHUNK, CHUNK)
        cps.append(pltpu.async_copy(bug.at[src], out_ug.at[uidx.at[j]], sem0))
        cps.append(pltpu.async_copy(big.at[src], out_ig.at[iidx.at[j]], sem1))
        cps.append(pltpu.async_copy(bum.at[src], out_um.at[uidx.at[j]], sem2))
        cps.append(pltpu.async_copy(bim.at[src], out_im.at[iidx.at[j]], sem3))
    for cp in cps:
        cp.wait()


@functools.cache
def _sc_unperm():
  return pl.kernel(
    _sc_unperm_body,
    out_type=[jax.ShapeDtypeStruct((BATCH, EMBED), jnp.float32)] * 4,
    mesh=plsc.VectorSubcoreMesh(core_axis_name="c", subcore_axis_name="s",
                                num_cores=NC, num_subcores=NS),
    scratch_types=[
        pltpu.VMEM((NCHUNK, CHUNK), jnp.int32),
        pltpu.VMEM((NCHUNK, CHUNK), jnp.int32),
        pltpu.VMEM((B_PER_W, EMBED), jnp.float32),
        pltpu.VMEM((B_PER_W, EMBED), jnp.float32),
        pltpu.VMEM((B_PER_W, EMBED), jnp.float32),
        pltpu.VMEM((B_PER_W, EMBED), jnp.float32),
        pltpu.SemaphoreType.DMA,
        pltpu.SemaphoreType.DMA,
        pltpu.SemaphoreType.DMA,
        pltpu.SemaphoreType.DMA,
    ],
    compiler_params=pltpu.CompilerParams(use_tc_tiling_on_sc=False),
  )


def _tc_mlp_body(ug, ig, um, im, w1a, w1b, b1, w2, b2, w3, b3, w4, b4,
                 wfg, wfh, bf, out_ref):
    dot = functools.partial(jnp.dot, preferred_element_type=jnp.float32)
    gmf = ug[...] * ig[...]
    h = jnp.maximum(dot(um[...], w1a[...]) + dot(im[...], w1b[...]) + b1[...], 0.0)
    h = jnp.maximum(dot(h, w2[...]) + b2[...], 0.0)
    h = jnp.maximum(dot(h, w3[...]) + b3[...], 0.0)
    h = jnp.maximum(dot(h, w4[...]) + b4[...], 0.0)
    logit = (jnp.sum(gmf * wfg[...], axis=1) + jnp.sum(h * wfh[...], axis=1)
             + bf[0])
    out_ref[...] = 1.0 / (1.0 + jnp.exp(-logit))


TCB = 4096


def kernel(user, item, ue_gmf, ie_gmf, ue_mlp, ie_mlp,
           W1, b1, W2, b2, W3, b3, W4, b4, Wf, bf):
    user = user.astype(jnp.int32)
    item = item.astype(jnp.int32)
    iota = jnp.arange(BATCH, dtype=jnp.int32)
    us, pu = lax.sort((user, iota), num_keys=1)
    uctl = _ring_control(us)
    sug, sum_ = _sc_gather_half()(*uctl, ue_gmf.T, ue_mlp.T)
    its, pi_ = lax.sort((item, iota), num_keys=1)
    ictl = _ring_control(its)
    sig, sim = _sc_gather_half()(*ictl, ie_gmf.T, ie_mlp.T)
    ug, ig, um, im = _sc_unperm()(
        pu.reshape(NW, NCHUNK, CHUNK), pi_.reshape(NW, NCHUNK, CHUNK),
        sug, sig, sum_, sim)

    weights = (W1[:EMBED], W1[EMBED:], b1.reshape(1, -1),
               W2, b2.reshape(1, -1), W3, b3.reshape(1, -1),
               W4, b4.reshape(1, -1),
               Wf[:EMBED].T, Wf[EMBED:].T)
    row_spec = pl.BlockSpec((TCB, EMBED), lambda i: (i, 0))
    wspecs = [pl.BlockSpec(w.shape, lambda i: (0, 0)) for w in weights]
    out = pl.pallas_call(
        _tc_mlp_body,
        grid=(BATCH // TCB,),
        in_specs=[row_spec] * 4 + wspecs
        + [pl.BlockSpec(memory_space=pltpu.SMEM)],
        out_specs=pl.BlockSpec((TCB,), lambda i: (i,)),
        out_shape=jax.ShapeDtypeStruct((BATCH,), jnp.float32),
    )(ug, ig, um, im, *weights, bf)
    return out
